# baseline (speedup 1.0000x reference)
import numpy as np
import jax
import jax.numpy as jnp
from jax.experimental import pallas as pl

_B = 32
_N = 16
_CUT = 3.5
_EPS = 1e-7
_NPAIR = (_N * (_N - 1)) // 2
_P = 128
_GROUPS = _N
_ROWS = _GROUPS * _P
_MOLS = 16

_pairs = [(j, k) for j in range(_N) for k in range(j + 1, _N)]
_Jn = np.array([p[0] for p in _pairs] + [0] * (_P - _NPAIR), np.int32)
_Kn = np.array([p[1] for p in _pairs] + [0] * (_P - _NPAIR), np.int32)
_SJ = np.zeros((_N, _P), np.float32)
_SJ[_Jn, np.arange(_P)] = 1.0
_SK = np.zeros((_N, _P), np.float32)
_SK[_Kn, np.arange(_P)] = 1.0
_SMASK16 = ((_Jn[None, :] != np.arange(_N)[:, None])
            & (_Kn[None, :] != np.arange(_N)[:, None])
            & (np.arange(_P) < _NPAIR)[None, :]).astype(np.float32)
_I16 = np.eye(_N, dtype=np.float32)


def _one_molecule(c3, c3t, zrow, eye, sj, sk, smask,
                  W0, b0, W1, b1, W2, b2, W3, b3, W4, b4, W5, b5, W6, b6):
    f32 = jnp.float32

    def dg(u, v, dims):
        return jax.lax.dot_general(u, v, (dims, ((), ())),
                                   preferred_element_type=f32)

    def dot(u, v):
        return dg(u, v, ((1,), (0,)))

    def dot_tl(u, v):
        return dg(u, v, ((0,), (0,)))

    dx = c3[:, 0:1] - c3t[0:1, :]
    dy = c3[:, 1:2] - c3t[1:2, :]
    dz = c3[:, 2:3] - c3t[2:3, :]
    d2 = dx * dx + dy * dy + dz * dz
    dist = jnp.sqrt(jnp.maximum(d2, 1e-12))

    sjv = sj[...]
    skv = sk[...]
    a = jnp.zeros((_N, _P), f32)
    b = jnp.zeros((_N, _P), f32)
    for j in range(_N):
        a = a + dist[:, j:j + 1] * sjv[j:j + 1, :]
        b = b + dist[:, j:j + 1] * skv[j:j + 1, :]
    c = jnp.sum(sjv * b, axis=0, keepdims=True)
    z_j = dot(zrow, sj[...])
    z_k = dot(zrow, sk[...])
    z_i = dg(eye[...], zrow, ((1,), (1,)))

    def carnot(x, y, w):
        return (x * x + y * y - w * w) / jnp.maximum(2.0 * x * y, 1e-10)

    ct_i = carnot(a, b, c)
    ct_j = carnot(a, c, b)
    ct_k = carnot(b, c, a)

    g0 = a + b + c
    g1 = a * b + a * c + b * c
    g2 = a * b * c
    gn = jnp.sqrt(g0 * g0 + g1 * g1 + g2 * g2) + _EPS
    c0 = (z_i + z_j + z_k) + 0.0 * a
    c1 = ct_i + ct_j + ct_k
    c2 = z_i * (z_j + z_k) + z_j * z_k - ct_i * (ct_j + ct_k) - ct_j * ct_k
    c3f = z_i * (ct_j + ct_k) + ct_i * (z_j + z_k) + z_j * ct_k + ct_j * z_k
    c4 = z_i * (z_j * z_k - ct_j * ct_k) - ct_i * (z_j * ct_k + ct_j * z_k)
    c5 = z_i * (z_j * ct_k + ct_j * z_k) + ct_i * (z_j * z_k - ct_j * ct_k)
    cn = jnp.sqrt(c0 * c0 + c1 * c1 + c2 * c2
                  + c3f * c3f + c4 * c4 + c5 * c5) + _EPS

    maps = [g0 / gn, g1 / gn, g2 / gn,
            c0 / cn, c1 / cn, c2 / cn, c3f / cn, c4 / cn, c5 / cn]

    fa = 0.5 * jnp.cos(np.pi / _CUT * a) + 0.5
    fb = 0.5 * jnp.cos(np.pi / _CUT * b) + 0.5
    w16 = jnp.where((a < _CUT) & (b < _CUT), fa * fb, 0.0) * smask[...]

    def flat_rows(mp):
        return jnp.concatenate([mp[i:i + 1, :] for i in range(_N)], axis=1)

    ft = jnp.concatenate(
        [flat_rows(mp) for mp in maps]
        + [jnp.zeros((16 - 9, _ROWS), f32)], axis=0)
    wv = flat_rows(w16)

    return ft, wv


def _fused_kernel(c_ref, ct_ref, z_ref, eye, sj, sk, smask,
                  W0, b0, W1, b1, W2, b2, W3, b3, W4, b4, W5, b5, W6, b6,
                  out_ref):
    f32 = jnp.float32

    def dot(u, v):
        return jax.lax.dot_general(u, v, (((1,), (0,)), ((), ())),
                                   preferred_element_type=f32)

    def dot_tl(u, v):
        return jax.lax.dot_general(u, v, (((0,), (0,)), ((), ())),
                                   preferred_element_type=f32)

    fts, wvs = [], []
    for mm in range(_MOLS):
        ftm, wvm = _one_molecule(c_ref[mm], ct_ref[mm], z_ref[mm],
                                 eye, sj, sk, smask,
                                 W0, b0, W1, b1, W2, b2, W3, b3, W4, b4,
                                 W5, b5, W6, b6)
        fts.append(ftm)
        wvs.append(wvm)
    ft = jnp.concatenate(fts, axis=1)

    x_res = jnp.tanh(dot_tl(ft, W0[...]) + b0[...])
    x1 = jnp.tanh(dot(x_res, W1[...]) + b1[...])
    xb1 = x1 + x_res
    h = jnp.tanh(dot(xb1, W2[...]) + b2[...])
    h = jnp.tanh(dot(h, W3[...]) + b3[...])
    h = jnp.tanh(dot(h, W4[...]) + b4[...])
    xb2 = h + xb1
    xb3 = jnp.tanh(dot(xb2, W5[...]) + b5[...])
    m = jnp.tanh(dot(xb3.astype(jnp.bfloat16), W6[...]) + b6[...])

    seg = jax.lax.broadcasted_iota(jnp.int32, (_GROUPS, _ROWS), 1) // _P
    row = jax.lax.broadcasted_iota(jnp.int32, (_GROUPS, _ROWS), 0)
    for mm in range(_MOLS):
        S = jnp.where(seg == row, wvs[mm], 0.0)
        out_ref[mm * _N:(mm + 1) * _N, :] = dot(
            S, m[mm * _ROWS:(mm + 1) * _ROWS, :])


def kernel(coordinates, species, W0, b0, W1, b1, W2, b2, W3, b3, W4, b4,
           W5, b5, W6, b6):
    zflat = species.astype(jnp.float32).reshape(_B, 1, _N)
    coordst = jnp.swapaxes(coordinates, 1, 2)
    W0p = jnp.concatenate([W0, jnp.zeros((16 - 9, 64), jnp.float32)], axis=0)

    statics = (jnp.asarray(_I16), jnp.asarray(_SJ), jnp.asarray(_SK),
               jnp.asarray(_SMASK16))
    weights = (W0p, b0.reshape(1, -1), W1, b1.reshape(1, -1),
               W2, b2.reshape(1, -1), W3, b3.reshape(1, -1),
               W4, b4.reshape(1, -1), W5, b5.reshape(1, -1),
               W6.astype(jnp.bfloat16), b6.reshape(1, -1))

    def full2(shape):
        return pl.BlockSpec(shape, lambda s: (0, 0))

    out = pl.pallas_call(
        _fused_kernel,
        grid=(_B // _MOLS,),
        in_specs=[pl.BlockSpec((_MOLS, _N, 3), lambda s: (s, 0, 0)),
                  pl.BlockSpec((_MOLS, 3, _N), lambda s: (s, 0, 0)),
                  pl.BlockSpec((_MOLS, 1, _N), lambda s: (s, 0, 0))]
                 + [full2(arr.shape) for arr in statics]
                 + [full2(arr.shape) for arr in weights],
        out_specs=pl.BlockSpec((_MOLS * _N, 256), lambda s: (s, 0)),
        out_shape=jax.ShapeDtypeStruct((_B * _N, 256), jnp.float32),
    )(coordinates, coordst, zflat, *statics, *weights)
    return out.reshape(_B, _N, 256)


# speedup vs baseline: 1.0281x; 1.0281x over previous
import numpy as np
import jax
import jax.numpy as jnp
from jax.experimental import pallas as pl

_B = 32
_N = 16
_CUT = 3.5
_EPS = 1e-7
_NPAIR = (_N * (_N - 1)) // 2
_P = 128
_GROUPS = _N
_ROWS = _GROUPS * _P
_MOLS = 16

_pairs = [(j, k) for j in range(_N) for k in range(j + 1, _N)]
_Jn = np.array([p[0] for p in _pairs] + [0] * (_P - _NPAIR), np.int32)
_Kn = np.array([p[1] for p in _pairs] + [0] * (_P - _NPAIR), np.int32)
_SJ = np.zeros((_N, _P), np.float32)
_SJ[_Jn, np.arange(_P)] = 1.0
_SK = np.zeros((_N, _P), np.float32)
_SK[_Kn, np.arange(_P)] = 1.0
_SMASK16 = ((_Jn[None, :] != np.arange(_N)[:, None])
            & (_Kn[None, :] != np.arange(_N)[:, None])
            & (np.arange(_P) < _NPAIR)[None, :]).astype(np.float32)
_I16 = np.eye(_N, dtype=np.float32)


def _one_molecule(c3, c3t, zrow, eye, sj, sk, smask,
                  W0, b0, W1, b1, W2, b2, W3, b3, W4, b4, W5, b5, W6, b6):
    f32 = jnp.float32

    def dg(u, v, dims):
        return jax.lax.dot_general(u, v, (dims, ((), ())),
                                   preferred_element_type=f32)

    def dot(u, v):
        return dg(u, v, ((1,), (0,)))

    def dot_tl(u, v):
        return dg(u, v, ((0,), (0,)))

    dx = c3[:, 0:1] - c3t[0:1, :]
    dy = c3[:, 1:2] - c3t[1:2, :]
    dz = c3[:, 2:3] - c3t[2:3, :]
    d2 = dx * dx + dy * dy + dz * dz
    dist = jnp.sqrt(jnp.maximum(d2, 1e-12))

    sjv = sj[...]
    skv = sk[...]
    a = jnp.zeros((_N, _P), f32)
    b = jnp.zeros((_N, _P), f32)
    for j in range(_N):
        a = a + dist[:, j:j + 1] * sjv[j:j + 1, :]
        b = b + dist[:, j:j + 1] * skv[j:j + 1, :]
    c = jnp.sum(sjv * b, axis=0, keepdims=True)
    z_j = dot(zrow, sj[...])
    z_k = dot(zrow, sk[...])
    z_i = dg(eye[...], zrow, ((1,), (1,)))

    def carnot(x, y, w):
        return (x * x + y * y - w * w) / jnp.maximum(2.0 * x * y, 1e-10)

    ct_i = carnot(a, b, c)
    ct_j = carnot(a, c, b)
    ct_k = carnot(b, c, a)

    g0 = a + b + c
    g1 = a * b + a * c + b * c
    g2 = a * b * c
    gn = jnp.sqrt(g0 * g0 + g1 * g1 + g2 * g2) + _EPS
    c0 = (z_i + z_j + z_k) + 0.0 * a
    c1 = ct_i + ct_j + ct_k
    c2 = z_i * (z_j + z_k) + z_j * z_k - ct_i * (ct_j + ct_k) - ct_j * ct_k
    c3f = z_i * (ct_j + ct_k) + ct_i * (z_j + z_k) + z_j * ct_k + ct_j * z_k
    c4 = z_i * (z_j * z_k - ct_j * ct_k) - ct_i * (z_j * ct_k + ct_j * z_k)
    c5 = z_i * (z_j * ct_k + ct_j * z_k) + ct_i * (z_j * z_k - ct_j * ct_k)
    cn = jnp.sqrt(c0 * c0 + c1 * c1 + c2 * c2
                  + c3f * c3f + c4 * c4 + c5 * c5) + _EPS

    maps = [g0 / gn, g1 / gn, g2 / gn,
            c0 / cn, c1 / cn, c2 / cn, c3f / cn, c4 / cn, c5 / cn]

    fa = 0.5 * jnp.cos(np.pi / _CUT * a) + 0.5
    fb = 0.5 * jnp.cos(np.pi / _CUT * b) + 0.5
    w16 = jnp.where((a < _CUT) & (b < _CUT), fa * fb, 0.0) * smask[...]

    def flat_rows(mp):
        return jnp.concatenate([mp[i:i + 1, :] for i in range(_N)], axis=1)

    ft = jnp.concatenate(
        [flat_rows(mp) for mp in maps]
        + [jnp.zeros((16 - 9, _ROWS), f32)], axis=0)
    wv = flat_rows(w16)

    return ft, wv


def _fused_kernel(c_ref, ct_ref, z_ref, eye, sj, sk, smask,
                  W0, b0, W1, b1, W2, b2, W3, b3, W4, b4, W5, b5, W6, b6,
                  out_ref):
    f32 = jnp.float32

    def dot(u, v):
        return jax.lax.dot_general(u, v, (((1,), (0,)), ((), ())),
                                   preferred_element_type=f32)

    def dot_tl(u, v):
        return jax.lax.dot_general(u, v, (((0,), (0,)), ((), ())),
                                   preferred_element_type=f32)

    fts, wvs = [], []
    for mm in range(_MOLS):
        ftm, wvm = _one_molecule(c_ref[mm], ct_ref[mm], z_ref[mm],
                                 eye, sj, sk, smask,
                                 W0, b0, W1, b1, W2, b2, W3, b3, W4, b4,
                                 W5, b5, W6, b6)
        fts.append(ftm)
        wvs.append(wvm)
    ft = jnp.concatenate(fts, axis=1)

    x_res = jnp.tanh(dot_tl(ft, W0[...]) + b0[...])
    x1 = jnp.tanh(dot(x_res, W1[...]) + b1[...])
    xb1 = x1 + x_res
    h = jnp.tanh(dot(xb1, W2[...]) + b2[...])
    h = jnp.tanh(dot(h, W3[...]) + b3[...])
    h = jnp.tanh(dot(h, W4[...]) + b4[...])
    xb2 = h + xb1
    xb3 = jnp.tanh(dot(xb2, W5[...]) + b5[...])
    m = jnp.tanh(dot(xb3, W6[...]) + b6[...])

    seg = jax.lax.broadcasted_iota(jnp.int32, (_GROUPS, _ROWS), 1) // _P
    row = jax.lax.broadcasted_iota(jnp.int32, (_GROUPS, _ROWS), 0)
    for mm in range(_MOLS):
        S = jnp.where(seg == row, wvs[mm], 0.0)
        out_ref[mm * _N:(mm + 1) * _N, :] = dot(
            S, m[mm * _ROWS:(mm + 1) * _ROWS, :])


def kernel(coordinates, species, W0, b0, W1, b1, W2, b2, W3, b3, W4, b4,
           W5, b5, W6, b6):
    zflat = species.astype(jnp.float32).reshape(_B, 1, _N)
    coordst = jnp.swapaxes(coordinates, 1, 2)
    W0p = jnp.concatenate([W0, jnp.zeros((16 - 9, 64), jnp.float32)], axis=0)

    statics = (jnp.asarray(_I16), jnp.asarray(_SJ), jnp.asarray(_SK),
               jnp.asarray(_SMASK16))
    weights = (W0p, b0.reshape(1, -1), W1, b1.reshape(1, -1),
               W2, b2.reshape(1, -1), W3, b3.reshape(1, -1),
               W4, b4.reshape(1, -1), W5, b5.reshape(1, -1),
               W6, b6.reshape(1, -1))

    def full2(shape):
        return pl.BlockSpec(shape, lambda s: (0, 0))

    out = pl.pallas_call(
        _fused_kernel,
        grid=(_B // _MOLS,),
        in_specs=[pl.BlockSpec((_MOLS, _N, 3), lambda s: (s, 0, 0)),
                  pl.BlockSpec((_MOLS, 3, _N), lambda s: (s, 0, 0)),
                  pl.BlockSpec((_MOLS, 1, _N), lambda s: (s, 0, 0))]
                 + [full2(arr.shape) for arr in statics]
                 + [full2(arr.shape) for arr in weights],
        out_specs=pl.BlockSpec((_MOLS * _N, 256), lambda s: (s, 0)),
        out_shape=jax.ShapeDtypeStruct((_B * _N, 256), jnp.float32),
    )(coordinates, coordst, zflat, *statics, *weights)
    return out.reshape(_B, _N, 256)
